# baseline (speedup 1.0000x reference)
import functools

import jax
import jax.numpy as jnp
from jax import lax
from jax.experimental import pallas as pl
from jax.experimental.pallas import tpu as pltpu
from jax.experimental.pallas import tpu_sc as plsc

B, D, T, K = 32, 64, 1024, 1024
BB = 2
TT = BB * T


def _vq_tc_body(x_ref, emb_ref, embm2_ref, dig_ref, idx_ref, loss_ref):
    b = pl.program_id(0)
    xb = jnp.concatenate([x_ref[i] for i in range(BB)], axis=1)
    emb = emb_ref[...]
    e2 = jnp.sum(emb * emb, axis=1)
    x2 = jnp.sum(xb * xb, axis=0)
    m2 = lax.dot_general(embm2_ref[...], xb, (((1,), (0,)), ((), ())),
                         preferred_element_type=jnp.float32)
    dist = (x2[None, :] + e2[:, None]) + m2
    minv = jnp.min(dist, axis=0)
    onehot = jnp.where(dist == minv[None, :], 1.0, 0.0)

    mom = lax.dot_general(dig_ref[...], onehot, (((0,), (0,)), ((), ())),
                          preferred_element_type=jnp.float32)
    cnt = mom[0]
    s1 = mom[1] * 256.0 + mom[2]
    s2 = (mom[3] * 65536.0 + mom[4] * 256.0) + mom[5]
    k1 = (s1 - jnp.sqrt(cnt * s2 - s1 * s1)) / cnt
    idx_ref[...] = k1.astype(jnp.int32).reshape(BB, 1, T)

    part = jnp.sum(minv)

    @pl.when(b == 0)
    def _():
        loss_ref[0, 0] = part

    @pl.when(b > 0)
    def _():
        loss_ref[0, 0] += part

    @pl.when(b == (B // BB) - 1)
    def _():
        loss_ref[0, 0] = loss_ref[0, 0] * (2.0 / (B * T * D))


_TCH = 512
_NTC = T // _TCH
_G16 = _TCH // 16

_sc_mesh = plsc.VectorSubcoreMesh(core_axis_name="c", subcore_axis_name="s")


@functools.partial(
    pl.kernel, mesh=_sc_mesh,
    compiler_params=pltpu.CompilerParams(needs_layout_passes=False),
    out_type=jax.ShapeDtypeStruct((B, D, T), jnp.float32),
    scratch_types=[
        pltpu.VMEM((K * D,), jnp.float32),
        pltpu.VMEM((T,), jnp.int32),
        pltpu.VMEM((D, _TCH), jnp.float32),
    ],
)
def _vq_sc_gather(embt_hbm, idx_hbm, out_hbm, emb_v, idx_v, out_v):
    nc = jax.lax.axis_size("c")
    wid = lax.axis_index("s") * nc + lax.axis_index("c")
    pltpu.sync_copy(embt_hbm, emb_v)
    pltpu.sync_copy(idx_hbm.at[wid], idx_v)
    for c in range(_NTC):
        @plsc.parallel_loop(0, _G16, unroll=4)
        def _(g):
            tbase = c * _TCH + g * 16
            kidx = idx_v[pl.ds(tbase, 16)]
            for d in range(D):
                out_v[d, pl.ds(g * 16, 16)] = plsc.load_gather(
                    emb_v, [kidx + (d * K)])
        pltpu.sync_copy(out_v, out_hbm.at[wid, :, pl.ds(c * _TCH, _TCH)])


@jax.jit
def kernel(x, embedding):
    embm2 = embedding * (-2.0)
    kv = lax.iota(jnp.int32, K)
    ksq = kv * kv
    digits = jnp.stack(
        [jnp.ones((K,), jnp.int32), kv // 256, kv % 256,
         ksq // 65536, (ksq % 65536) // 256, ksq % 256],
        axis=1).astype(jnp.float32)
    idx3, loss = pl.pallas_call(
        _vq_tc_body,
        grid=(B // BB,),
        in_specs=[
            pl.BlockSpec((BB, D, T), lambda b: (b, 0, 0)),
            pl.BlockSpec((K, D), lambda b: (0, 0)),
            pl.BlockSpec((K, D), lambda b: (0, 0)),
            pl.BlockSpec((K, 6), lambda b: (0, 0)),
        ],
        out_specs=[
            pl.BlockSpec((BB, 1, T), lambda b: (b, 0, 0)),
            pl.BlockSpec(memory_space=pltpu.SMEM, block_shape=(1, 1),
                         index_map=lambda b: (0, 0)),
        ],
        out_shape=[
            jax.ShapeDtypeStruct((B, 1, T), jnp.int32),
            jax.ShapeDtypeStruct((1, 1), jnp.float32),
        ],
    )(x, embedding, embm2, digits)
    idx = idx3.reshape(B, T)
    values = _vq_sc_gather(embedding.T.reshape(D * K), idx)
    return values, idx, loss[0, 0]


# speedup vs baseline: 1.6992x; 1.6992x over previous
import jax
import jax.numpy as jnp
from jax import lax
from jax.experimental import pallas as pl
from jax.experimental.pallas import tpu as pltpu

B, D, T, K = 32, 64, 1024, 1024
BB = 4
TT = BB * T


def _vq_body(x_ref, emb_ref, embm2_ref, embaug_ref, idx_ref, val_ref, loss_ref):
    b = pl.program_id(0)
    xb = jnp.concatenate([x_ref[i] for i in range(BB)], axis=1)
    emb = emb_ref[...]
    e2 = jnp.sum(emb * emb, axis=1)
    x2 = jnp.sum(xb * xb, axis=0)
    m2 = lax.dot_general(embm2_ref[...], xb, (((1,), (0,)), ((), ())),
                         preferred_element_type=jnp.float32)
    dist = (x2[None, :] + e2[:, None]) + m2
    minv = jnp.min(dist, axis=0)
    onehot = jnp.where(dist == minv[None, :], 1.0, 0.0)

    aug = lax.dot_general(embaug_ref[...], onehot, (((0,), (0,)), ((), ())),
                          preferred_element_type=jnp.float32)
    vals = aug[:D]
    cnt = aug[D]
    s1 = aug[D + 1] * 256.0 + aug[D + 2]
    s2 = (aug[D + 3] * 65536.0 + aug[D + 4] * 256.0) + aug[D + 5]
    k1 = (s1 - jnp.sqrt(cnt * s2 - s1 * s1)) / cnt
    idx = k1.astype(jnp.int32)
    idx_ref[...] = idx.reshape(BB, 1, T)

    for i in range(BB):
        val_ref[i] = vals[:, i * T:(i + 1) * T]

    diff = xb - vals
    part = jnp.sum(diff * diff)

    @pl.when(jnp.max(cnt) > 1.5)
    def _():
        kiota = lax.broadcasted_iota(jnp.int32, (K, TT), 0)
        onehot2 = jnp.where(kiota == idx[None, :], 1.0, 0.0)
        vals2 = lax.dot_general(emb, onehot2, (((0,), (0,)), ((), ())),
                                preferred_element_type=jnp.float32)
        for i in range(BB):
            val_ref[i] = vals2[:, i * T:(i + 1) * T]

    @pl.when(b == 0)
    def _():
        loss_ref[0, 0] = part

    @pl.when(b > 0)
    def _():
        loss_ref[0, 0] += part

    @pl.when(b == (B // BB) - 1)
    def _():
        loss_ref[0, 0] = loss_ref[0, 0] * (2.0 / (B * T * D))


@jax.jit
def kernel(x, embedding):
    embm2 = embedding * (-2.0)
    kv = lax.iota(jnp.int32, K)
    ksq = kv * kv
    digits = jnp.stack(
        [jnp.ones((K,), jnp.int32), kv // 256, kv % 256,
         ksq // 65536, (ksq % 65536) // 256, ksq % 256],
        axis=1).astype(jnp.float32)
    embaug = jnp.concatenate([embedding, digits], axis=1)
    idx3, values, loss = pl.pallas_call(
        _vq_body,
        grid=(B // BB,),
        in_specs=[
            pl.BlockSpec((BB, D, T), lambda b: (b, 0, 0)),
            pl.BlockSpec((K, D), lambda b: (0, 0)),
            pl.BlockSpec((K, D), lambda b: (0, 0)),
            pl.BlockSpec((K, D + 6), lambda b: (0, 0)),
        ],
        out_specs=[
            pl.BlockSpec((BB, 1, T), lambda b: (b, 0, 0)),
            pl.BlockSpec((BB, D, T), lambda b: (b, 0, 0)),
            pl.BlockSpec(memory_space=pltpu.SMEM, block_shape=(1, 1),
                         index_map=lambda b: (0, 0)),
        ],
        out_shape=[
            jax.ShapeDtypeStruct((B, 1, T), jnp.int32),
            jax.ShapeDtypeStruct((B, D, T), jnp.float32),
            jax.ShapeDtypeStruct((1, 1), jnp.float32),
        ],
    )(x, embedding, embm2, embaug)
    return values, idx3.reshape(B, T), loss[0, 0]
